# baseline (speedup 1.0000x reference)
import functools

import numpy as np
import jax
import jax.numpy as jnp
from jax import lax
from jax.experimental import pallas as pl
from jax.experimental.pallas import tpu as pltpu
from jax.experimental.pallas import tpu_sc as plsc

NUM_WORKERS = 32
LANES = 16
CH = 4


def _build_sc_kernel(B, S, D, NB):
    SP = S + 1
    P = ((SP + 7) // 8) * 8
    S_PER_W = S // NUM_WORKERS
    T = S_PER_W // CH
    GI = B * CH
    SLAB = CH * B * D
    mesh = plsc.VectorSubcoreMesh(core_axis_name="c", subcore_axis_name="s")

    @functools.partial(
        pl.kernel,
        mesh=mesh,
        out_type=jax.ShapeDtypeStruct((SP * B * D,), jnp.float32),
        scratch_types=[
            pltpu.VMEM((T * GI + 8,), jnp.int32),
            pltpu.VMEM((T * 8 + 8,), jnp.int32),
            pltpu.VMEM((GI, D), jnp.float32),
            pltpu.VMEM((GI, D), jnp.float32),
            pltpu.VMEM((CH, D), jnp.float32),
            pltpu.VMEM((CH, D), jnp.float32),
            pltpu.VMEM((SLAB,), jnp.float32),
            pltpu.VMEM((SLAB,), jnp.float32),
            pltpu.VMEM((D,), jnp.float32),
            pltpu.SemaphoreType.DMA,
            pltpu.SemaphoreType.DMA,
            pltpu.SemaphoreType.DMA,
            pltpu.SemaphoreType.DMA,
            pltpu.SemaphoreType.DMA,
            pltpu.SemaphoreType.DMA,
        ],
    )
    def sc_embed(idx_hbm, table_hbm, pos_hbm, cls_hbm, out_hbm,
                 gidx_v, pidx_v, rows0, rows1, posb0, posb1, slab0, slab1,
                 cls_v, sg0, sg1, sp0, sp1, so0, so1):
        PIDX_OFF = NUM_WORKERS * T * GI + 8
        wid = lax.axis_index("c") * 16 + lax.axis_index("s")
        s0 = wid * S_PER_W
        rows = (rows0, rows1)
        posb = (posb0, posb1)
        slab = (slab0, slab1)
        sgs = (sg0, sg1)
        sps = (sp0, sp1)
        sos = (so0, so1)

        pltpu.sync_copy(idx_hbm.at[pl.ds(wid * T * GI, T * GI)],
                        gidx_v.at[pl.ds(0, T * GI)])
        pltpu.sync_copy(idx_hbm.at[pl.ds(PIDX_OFF + wid * T * 8, T * 8)],
                        pidx_v.at[pl.ds(0, T * 8)])

        @pl.when(wid == 0)
        def _():
            pltpu.sync_copy(cls_hbm, cls_v)

        def gather_start(t, k):
            pltpu.async_copy(
                table_hbm.at[gidx_v.at[pl.ds(t * GI, GI)]], rows[k], sgs[k])
            pltpu.async_copy(
                pos_hbm.at[pidx_v.at[pl.ds(t * 8, CH)]], posb[k], sps[k])

        def gather_wait(k):
            pltpu.make_async_copy(table_hbm.at[pl.ds(0, GI)],
                                  rows[k], sgs[k]).wait()
            pltpu.make_async_copy(pos_hbm.at[pl.ds(0, CH)],
                                  posb[k], sps[k]).wait()

        def out_start(t, k):
            off = (s0 + t * CH) * B * D
            pltpu.async_copy(slab[k], out_hbm.at[pl.ds(off, SLAB)], sos[k])

        def out_wait(k):
            pltpu.make_async_copy(slab[k], out_hbm.at[pl.ds(0, SLAB)],
                                  sos[k]).wait()

        def add_interleave(rk, sk):
            for r in range(CH):
                for lb in range(NB):
                    for v in range(128 // LANES):
                        l = lb * 128 + v * LANES
                        pv = posb[rk][r, pl.ds(l, LANES)]
                        for b in range(B):
                            o = r * B * D + lb * B * 128 + b * 128 + v * LANES
                            slab[sk][pl.ds(o, LANES)] = (
                                rows[rk][b * CH + r, pl.ds(l, LANES)] + pv)

        gather_start(0, 0)

        @pl.loop(0, T, step=2)
        def _(tt):
            for kk in range(2):
                t = tt + kk
                rk = kk

                @pl.when(t + 1 < T)
                def _():
                    gather_start(t + 1, 1 - rk)

                gather_wait(rk)

                @pl.when(t >= 2)
                def _():
                    out_wait(kk)

                add_interleave(rk, kk)

                if kk == 0:
                    @pl.when((wid == 0) & (t == 0))
                    def _():
                        for lb in range(NB):
                            for v in range(128 // LANES):
                                l = lb * 128 + v * LANES
                                cv = cls_v[pl.ds(l, LANES)]
                                for b in range(B):
                                    o = lb * B * 128 + b * 128 + v * LANES
                                    slab[kk][pl.ds(o, LANES)] = cv

                out_start(t, kk)

        out_wait(0)
        out_wait(1)

        @pl.when(wid == NUM_WORKERS - 1)
        def _():
            pltpu.sync_copy(idx_hbm.at[pl.ds(NUM_WORKERS * T * GI, 8)],
                            gidx_v.at[pl.ds(0, 8)])
            pltpu.async_copy(table_hbm.at[gidx_v.at[pl.ds(0, 8)]],
                             rows0.at[pl.ds(0, 8)], sg0).wait()
            pltpu.sync_copy(
                idx_hbm.at[pl.ds(PIDX_OFF + NUM_WORKERS * T * 8, 8)],
                pidx_v.at[pl.ds(0, 8)])
            pltpu.async_copy(pos_hbm.at[pidx_v.at[pl.ds(0, CH)]],
                             posb0, sp0).wait()
            for lb in range(NB):
                for v in range(128 // LANES):
                    l = lb * 128 + v * LANES
                    pv = posb0[0, pl.ds(l, LANES)]
                    for b in range(B):
                        o = lb * B * 128 + b * 128 + v * LANES
                        slab0[pl.ds(o, LANES)] = (
                            rows0[b, pl.ds(l, LANES)] + pv)
            pltpu.sync_copy(slab0.at[pl.ds(0, B * D)],
                            out_hbm.at[pl.ds(S * B * D, B * D)])

    return sc_embed


def kernel(input, table, pos_embeds, cls):
    B, S = input.shape
    D = table.shape[1]
    SP = S + 1
    P = ((SP + 7) // 8) * 8
    NB = D // 128
    S_PER_W = S // NUM_WORKERS
    T = S_PER_W // CH
    GI = B * CH
    n_items = S // CH
    pvals = np.concatenate([
        np.clip(np.arange(n_items, dtype=np.int32)[:, None] * CH
                + np.arange(8, dtype=np.int32)[None, :] - 1,
                0, S - 1).reshape(-1),
        np.full((8,), S - 1, np.int32)])
    w_, c_, b_, r_ = np.meshgrid(
        np.arange(NUM_WORKERS), np.arange(T), np.arange(B), np.arange(CH),
        indexing="ij")
    j_ = (w_ * S_PER_W + c_ * CH + r_).reshape(-1)
    bflat = b_.reshape(-1)
    gperm = np.where(j_ == 0, 0, 1 + bflat * S + (j_ - 1)).astype(np.int32)
    tailperm = np.concatenate([
        1 + np.arange(B, dtype=np.int32) * S + (S - 1),
        np.zeros((8 - B,), np.int32)])
    pperm = 1 + B * S + np.arange(pvals.size, dtype=np.int32)
    perm = jnp.asarray(np.concatenate([gperm, tailperm, pperm]))
    source = jnp.concatenate(
        [jnp.zeros((1,), jnp.int32), input.reshape(-1), jnp.asarray(pvals)])
    idx_combined = jnp.take(source, perm)
    pos2d = pos_embeds.reshape(S, D)
    cls1d = cls.reshape(D)
    sc = _build_sc_kernel(B, S, D, NB)
    out_flat = sc(idx_combined, table, pos2d, cls1d)
    return (out_flat.reshape(SP, NB, B, 128)
            .transpose(2, 0, 1, 3)
            .reshape(B, SP, D))


# speedup vs baseline: 1.4318x; 1.4318x over previous
import functools

import numpy as np
import jax
import jax.numpy as jnp
from jax import lax
from jax.experimental import pallas as pl
from jax.experimental.pallas import tpu as pltpu
from jax.experimental.pallas import tpu_sc as plsc

NUM_WORKERS = 32
LANES = 16
CH = 4


def _build_sc_kernel(B, S, D, NB):
    SP = S + 1
    P = ((SP + 7) // 8) * 8
    S_PER_W = S // NUM_WORKERS
    T = S_PER_W // CH
    GI = B * CH
    SLAB = CH * B * D
    mesh = plsc.VectorSubcoreMesh(core_axis_name="c", subcore_axis_name="s")

    @functools.partial(
        pl.kernel,
        mesh=mesh,
        out_type=jax.ShapeDtypeStruct((SP * B * D,), jnp.float32),
        scratch_types=[
            pltpu.VMEM((T * GI + 8,), jnp.int32),
            pltpu.VMEM((T * 8 + 8,), jnp.int32),
            pltpu.VMEM((GI, D), jnp.float32),
            pltpu.VMEM((GI, D), jnp.float32),
            pltpu.VMEM((CH, D), jnp.float32),
            pltpu.VMEM((CH, D), jnp.float32),
            pltpu.VMEM((SLAB,), jnp.float32),
            pltpu.VMEM((SLAB,), jnp.float32),
            pltpu.VMEM((D,), jnp.float32),
            pltpu.VMEM((8, D), jnp.float32),
            pltpu.VMEM((8, D), jnp.float32),
            pltpu.SemaphoreType.DMA,
            pltpu.SemaphoreType.DMA,
            pltpu.SemaphoreType.DMA,
            pltpu.SemaphoreType.DMA,
            pltpu.SemaphoreType.DMA,
            pltpu.SemaphoreType.DMA,
            pltpu.SemaphoreType.DMA,
        ],
    )
    def sc_embed(idx_hbm, table_hbm, pos_hbm, cls_hbm, out_hbm,
                 gidx_v, pidx_v, rows0, rows1, posb0, posb1, slab0, slab1,
                 cls_v, trows, tpos, sg0, sg1, sp0, sp1, so0, so1, st):
        PIDX_OFF = NUM_WORKERS * T * GI + 8
        wid = lax.axis_index("c") * 16 + lax.axis_index("s")
        s0 = wid * S_PER_W
        rows = (rows0, rows1)
        posb = (posb0, posb1)
        slab = (slab0, slab1)
        sgs = (sg0, sg1)
        sps = (sp0, sp1)
        sos = (so0, so1)

        pltpu.sync_copy(idx_hbm.at[pl.ds(wid * T * GI, T * GI)],
                        gidx_v.at[pl.ds(0, T * GI)])
        pltpu.sync_copy(idx_hbm.at[pl.ds(PIDX_OFF + wid * T * 8, T * 8)],
                        pidx_v.at[pl.ds(0, T * 8)])

        @pl.when(wid == 0)
        def _():
            pltpu.sync_copy(cls_hbm, cls_v)

        @pl.when(wid == NUM_WORKERS - 1)
        def _():
            pltpu.sync_copy(idx_hbm.at[pl.ds(NUM_WORKERS * T * GI, 8)],
                            gidx_v.at[pl.ds(T * GI, 8)])
            pltpu.sync_copy(
                idx_hbm.at[pl.ds(PIDX_OFF + NUM_WORKERS * T * 8, 8)],
                pidx_v.at[pl.ds(T * 8, 8)])
            pltpu.async_copy(table_hbm.at[gidx_v.at[pl.ds(T * GI, 8)]],
                             trows, st)
            pltpu.async_copy(pos_hbm.at[pidx_v.at[pl.ds(T * 8, 8)]],
                             tpos, st)

        def gather_start(t, k):
            pltpu.async_copy(
                table_hbm.at[gidx_v.at[pl.ds(t * GI, GI)]], rows[k], sgs[k])
            pltpu.async_copy(
                pos_hbm.at[pidx_v.at[pl.ds(t * 8, CH)]], posb[k], sps[k])

        def gather_wait(k):
            pltpu.make_async_copy(table_hbm.at[pl.ds(0, GI)],
                                  rows[k], sgs[k]).wait()
            pltpu.make_async_copy(pos_hbm.at[pl.ds(0, CH)],
                                  posb[k], sps[k]).wait()

        def out_start(t, k):
            off = (s0 + t * CH) * B * D
            pltpu.async_copy(slab[k], out_hbm.at[pl.ds(off, SLAB)], sos[k])

        def out_wait(k):
            pltpu.make_async_copy(slab[k], out_hbm.at[pl.ds(0, SLAB)],
                                  sos[k]).wait()

        def add_interleave(rk, sk):
            for r in range(CH):
                for lb in range(NB):
                    for v in range(128 // LANES):
                        l = lb * 128 + v * LANES
                        pv = posb[rk][r, pl.ds(l, LANES)]
                        for b in range(B):
                            o = r * B * D + lb * B * 128 + b * 128 + v * LANES
                            slab[sk][pl.ds(o, LANES)] = (
                                rows[rk][b * CH + r, pl.ds(l, LANES)] + pv)

        gather_start(0, 0)

        @pl.loop(0, T, step=2)
        def _(tt):
            for kk in range(2):
                t = tt + kk
                rk = kk

                @pl.when(t + 1 < T)
                def _():
                    gather_start(t + 1, 1 - rk)

                gather_wait(rk)

                @pl.when(t >= 2)
                def _():
                    out_wait(kk)

                add_interleave(rk, kk)

                if kk == 0:
                    @pl.when((wid == 0) & (t == 0))
                    def _():
                        for lb in range(NB):
                            for v in range(128 // LANES):
                                l = lb * 128 + v * LANES
                                cv = cls_v[pl.ds(l, LANES)]
                                for b in range(B):
                                    o = lb * B * 128 + b * 128 + v * LANES
                                    slab[kk][pl.ds(o, LANES)] = cv

                out_start(t, kk)

        out_wait(0)
        out_wait(1)

        @pl.when(wid == NUM_WORKERS - 1)
        def _():
            pltpu.make_async_copy(table_hbm.at[pl.ds(0, 8)], trows,
                                  st).wait()
            pltpu.make_async_copy(pos_hbm.at[pl.ds(0, 8)], tpos, st).wait()
            for lb in range(NB):
                for v in range(128 // LANES):
                    l = lb * 128 + v * LANES
                    pv = tpos[0, pl.ds(l, LANES)]
                    for b in range(B):
                        o = lb * B * 128 + b * 128 + v * LANES
                        slab0[pl.ds(o, LANES)] = (
                            trows[b, pl.ds(l, LANES)] + pv)
            pltpu.sync_copy(slab0.at[pl.ds(0, B * D)],
                            out_hbm.at[pl.ds(S * B * D, B * D)])

    return sc_embed


def kernel(input, table, pos_embeds, cls):
    B, S = input.shape
    D = table.shape[1]
    SP = S + 1
    P = ((SP + 7) // 8) * 8
    NB = D // 128
    S_PER_W = S // NUM_WORKERS
    T = S_PER_W // CH
    GI = B * CH
    n_items = S // CH
    sidx = jnp.zeros((B, SP), jnp.int32).at[:, 1:].set(input)
    gmain = (sidx[:, :S]
             .reshape(B, NUM_WORKERS, S_PER_W // CH, CH)
             .transpose(1, 2, 0, 3)
             .reshape(-1))
    gtail = jnp.concatenate([sidx[:, S], jnp.zeros((8 - B,), jnp.int32)])
    pvals = np.concatenate([
        np.clip(np.arange(n_items, dtype=np.int32)[:, None] * CH
                + np.arange(8, dtype=np.int32)[None, :] - 1,
                0, S - 1).reshape(-1),
        np.full((8,), S - 1, np.int32)])
    idx_combined = jnp.concatenate([gmain, gtail, jnp.asarray(pvals)])
    pos2d = pos_embeds.reshape(S, D)
    cls1d = cls.reshape(D)
    sc = _build_sc_kernel(B, S, D, NB)
    out_flat = sc(idx_combined, table, pos2d, cls1d)
    return (out_flat.reshape(SP, NB, B, 128)
            .transpose(2, 0, 1, 3)
            .reshape(B, SP, D))
